# Initial kernel scaffold; baseline (speedup 1.0000x reference)
import jax
import jax.numpy as jnp
from jax.experimental import pallas as pl


def kernel(hidden_states, pos_table):
    raise NotImplementedError("write your pallas kernel here")


# speedup vs baseline: 1.3650x; 1.3650x over previous
import jax
import jax.numpy as jnp
from jax.experimental import pallas as pl

MAX_POS_ = 2048
HIDDEN_ = 768
BATCH_ = 4

BP = 256


def _add_body(hid_ref, pos_ref, out_ref):
    out_ref[...] = hid_ref[...] + pos_ref[...]


def kernel(hidden_states, pos_table):
    n_pos_blocks = MAX_POS_ // BP
    grid = (n_pos_blocks, BATCH_)
    return pl.pallas_call(
        _add_body,
        grid=grid,
        in_specs=[
            pl.BlockSpec((1, BP, HIDDEN_), lambda i, b: (b, i, 0)),
            pl.BlockSpec((BP, HIDDEN_), lambda i, b: (i, 0)),
        ],
        out_specs=pl.BlockSpec((1, BP, HIDDEN_), lambda i, b: (b, i, 0)),
        out_shape=jax.ShapeDtypeStruct((BATCH_, MAX_POS_, HIDDEN_), jnp.float32),
    )(hidden_states, pos_table)
